# baseline (speedup 1.0000x reference)
import functools

import jax
import jax.numpy as jnp
from jax import lax
from jax.experimental import pallas as pl
from jax.experimental.pallas import tpu as pltpu
from jax.experimental.pallas import tpu_sc as plsc

N = 10000
D = 128
H = 128
A = 32
E = 320000

NC = 2
NS = 16
L = 16
NW = NC * NS
CPT = H // NW

EPT = E // NW
CHUNK = 2000
NCH_A = EPT // CHUNK
NCH_B = E // CHUNK
GROUPS = CHUNK // L
UNROLL = 25

NB = 2048
GRID = (N + NB - 1) // NB

f32 = jnp.float32


def _stage1_body(x_ref, we_ref, be_ref, wg_ref, asr_ref, adr_ref,
                 localt_ref, h3_ref, as_ref, ad_ref):
    rowT = (((1,), (1,)), ((), ()))
    colT = (((1,), (0,)), ((), ()))
    localt = jnp.tanh(lax.dot_general(we_ref[...], x_ref[...], rowT,
                                      preferred_element_type=f32) + be_ref[...])
    ht = lax.dot_general(wg_ref[...], localt, colT, preferred_element_type=f32)
    localt_ref[...] = localt
    as_ref[...] = lax.dot_general(asr_ref[...], ht, colT,
                                  preferred_element_type=f32)
    ad_ref[...] = lax.dot_general(adr_ref[...], ht, colT,
                                  preferred_element_type=f32)
    for g in range(NW):
        h3_ref[g] = ht[CPT * g:CPT * (g + 1), :]


def _stage3_body(localt_ref, s3_ref, den_ref, bg3_ref, waffl_ref, waffr3_ref,
                 ba_ref, wact_ref, bact_ref, wval_ref, bval_ref, a_ref, v_ref):
    colT = (((1,), (0,)), ((), ()))
    den = jnp.sum(den_ref[:, 0, :], axis=0)[None, None, :]
    intra3 = jnp.tanh(s3_ref[...] / (den + 1e-16) + bg3_ref[...])
    intra = intra3.reshape(H, intra3.shape[2])
    aff_r = lax.dot_general(waffr3_ref[...], intra, colT,
                            preferred_element_type=f32)
    hidt = jnp.tanh(
        lax.dot_general(waffl_ref[...], localt_ref[...], colT,
                        preferred_element_type=f32)
        + aff_r + ba_ref[...])
    logits = lax.dot_general(hidt, wact_ref[...], (((0,), (1,)), ((), ())),
                             preferred_element_type=f32) + bact_ref[...]
    mx = jnp.max(logits, axis=1, keepdims=True)
    lse = jnp.log(jnp.sum(jnp.exp(logits - mx), axis=1, keepdims=True))
    a_ref[...] = logits - mx - lse
    v_ref[...] = lax.dot_general(hidt, wval_ref[...], (((0,), (1,)), ((), ())),
                                 preferred_element_type=f32) + bval_ref[...]


def _phase_a_body(src_hbm, dst_hbm, as_hbm, ad_hbm,
                  w_out, den_out,
                  src_v, dst_v, w_v, asv, adv, den_v):
    cid = lax.axis_index("c")
    sid = lax.axis_index("s")
    wid = sid * NC + cid
    base = wid * EPT

    pltpu.sync_copy(as_hbm, asv)
    pltpu.sync_copy(ad_hbm, adv)

    def _zden(i, _):
        den_v[pl.ds(i * L, L)] = jnp.zeros((L,), f32)
        return 0
    lax.fori_loop(0, N // L, _zden, 0)

    def _chunk(ci, _):
        pltpu.sync_copy(src_hbm.at[pl.ds(base + ci * CHUNK, CHUNK)], src_v)
        pltpu.sync_copy(dst_hbm.at[pl.ds(base + ci * CHUNK, CHUNK)], dst_v)

        def _group(g, _):
            off = g * L
            src16 = src_v[pl.ds(off, L)]
            dst16 = dst_v[pl.ds(off, L)]
            e = plsc.load_gather(asv, [src16]) + plsc.load_gather(adv, [dst16])
            e = jnp.where(e > 0, e, 0.2 * e)
            w = jnp.exp(e)
            plsc.addupdate_scatter(den_v, [dst16], w)
            w_v[pl.ds(off, L)] = w
            return 0
        lax.fori_loop(0, GROUPS, _group, 0)
        pltpu.sync_copy(w_v, w_out.at[wid, ci])
        return 0
    lax.fori_loop(0, NCH_A, _chunk, 0)

    pltpu.sync_copy(den_v, den_out.at[wid, 0])


def _phase_b_body(w_hbm, src_hbm, dst_hbm, h3_hbm,
                  s_out,
                  wa, sra, dsa, wb, srb, dsb,
                  h0, h1, h2, h3, s0, s1, s2, s3,
                  ga0, ga1, ga2, gb0, gb1, gb2):
    cid = lax.axis_index("c")
    sid = lax.axis_index("s")
    wid = sid * NC + cid
    hc = (h0, h1, h2, h3)
    sc = (s0, s1, s2, s3)

    for c in range(CPT):
        pltpu.sync_copy(h3_hbm.at[wid, c], hc[c])

    def _zs(i, _):
        z = jnp.zeros((L,), f32)
        for c in range(CPT):
            sc[c][pl.ds(i * L, L)] = z
        return 0
    lax.fori_loop(0, N // L, _zs, 0)

    def _issue(ci, w_b, s_b, d_b, sems):
        pltpu.async_copy(w_hbm.at[ci // NCH_A, ci % NCH_A], w_b, sems[0])
        pltpu.async_copy(src_hbm.at[pl.ds(ci * CHUNK, CHUNK)], s_b, sems[1])
        pltpu.async_copy(dst_hbm.at[pl.ds(ci * CHUNK, CHUNK)], d_b, sems[2])

    def _wait(ci, w_b, s_b, d_b, sems):
        pltpu.make_async_copy(w_hbm.at[ci // NCH_A, ci % NCH_A],
                              w_b, sems[0]).wait()
        pltpu.make_async_copy(src_hbm.at[pl.ds(ci * CHUNK, CHUNK)],
                              s_b, sems[1]).wait()
        pltpu.make_async_copy(dst_hbm.at[pl.ds(ci * CHUNK, CHUNK)],
                              d_b, sems[2]).wait()

    def _process(w_b, s_b, d_b):
        def _group(g, _):
            for u in range(UNROLL):
                off = (g * UNROLL + u) * L
                w16 = w_b[pl.ds(off, L)]
                src16 = s_b[pl.ds(off, L)]
                dst16 = d_b[pl.ds(off, L)]
                for c in range(CPT):
                    hv = plsc.load_gather(hc[c], [src16])
                    plsc.addupdate_scatter(sc[c], [dst16], w16 * hv)
            return 0
        lax.fori_loop(0, GROUPS // UNROLL, _group, 0)

    sems_a = (ga0, ga1, ga2)
    sems_b = (gb0, gb1, gb2)
    _issue(0, wa, sra, dsa, sems_a)

    def _pair(k, _):
        ca = 2 * k
        cb = 2 * k + 1
        _wait(ca, wa, sra, dsa, sems_a)
        _issue(cb, wb, srb, dsb, sems_b)
        _process(wa, sra, dsa)
        _wait(cb, wb, srb, dsb, sems_b)

        @pl.when(k < NCH_B // 2 - 1)
        def _():
            _issue(ca + 2, wa, sra, dsa, sems_a)
        _process(wb, srb, dsb)
        return 0
    lax.fori_loop(0, NCH_B // 2, _pair, 0)

    for c in range(CPT):
        pltpu.sync_copy(sc[c], s_out.at[wid, c])


def _phase_a_call():
    return functools.partial(
        pl.kernel,
        out_type=[jax.ShapeDtypeStruct((NW, NCH_A, CHUNK), f32),
                  jax.ShapeDtypeStruct((NW, 1, N), f32)],
        mesh=plsc.VectorSubcoreMesh(core_axis_name="c", subcore_axis_name="s",
                                    num_cores=NC, num_subcores=NS),
        scratch_types=[
            pltpu.VMEM((CHUNK,), jnp.int32),
            pltpu.VMEM((CHUNK,), jnp.int32),
            pltpu.VMEM((CHUNK,), f32),
            pltpu.VMEM((N,), f32),
            pltpu.VMEM((N,), f32),
            pltpu.VMEM((N,), f32),
        ],
        compiler_params=pltpu.CompilerParams(needs_layout_passes=False),
    )


def _phase_b_call():
    return functools.partial(
        pl.kernel,
        out_type=[jax.ShapeDtypeStruct((NW, CPT, N), f32)],
        mesh=plsc.VectorSubcoreMesh(core_axis_name="c", subcore_axis_name="s",
                                    num_cores=NC, num_subcores=NS),
        scratch_types=[
            pltpu.VMEM((CHUNK,), f32),
            pltpu.VMEM((CHUNK,), jnp.int32),
            pltpu.VMEM((CHUNK,), jnp.int32),
            pltpu.VMEM((CHUNK,), f32),
            pltpu.VMEM((CHUNK,), jnp.int32),
            pltpu.VMEM((CHUNK,), jnp.int32),
            pltpu.VMEM((N,), f32),
            pltpu.VMEM((N,), f32),
            pltpu.VMEM((N,), f32),
            pltpu.VMEM((N,), f32),
            pltpu.VMEM((N,), f32),
            pltpu.VMEM((N,), f32),
            pltpu.VMEM((N,), f32),
            pltpu.VMEM((N,), f32),
            pltpu.SemaphoreType.DMA,
            pltpu.SemaphoreType.DMA,
            pltpu.SemaphoreType.DMA,
            pltpu.SemaphoreType.DMA,
            pltpu.SemaphoreType.DMA,
            pltpu.SemaphoreType.DMA,
        ],
        compiler_params=pltpu.CompilerParams(needs_layout_passes=False),
    )


def kernel(x, edge_index, W_emb, b_emb, W_gat, att_src, att_dst, b_gat,
           W_aff, b_aff, W_act, b_act, W_val, b_val):
    full = lambda bs: pl.BlockSpec(bs, lambda i: (0,) * len(bs))

    localt, h3, a_s, a_d = pl.pallas_call(
        _stage1_body,
        grid=(GRID,),
        in_specs=[
            pl.BlockSpec((NB, D), lambda i: (i, 0)),
            full((H, D)), full((H, 1)), full((H, H)),
            full((1, H)), full((1, H)),
        ],
        out_specs=[
            pl.BlockSpec((H, NB), lambda i: (0, i)),
            pl.BlockSpec((NW, CPT, NB), lambda i: (0, 0, i)),
            pl.BlockSpec((1, NB), lambda i: (0, i)),
            pl.BlockSpec((1, NB), lambda i: (0, i)),
        ],
        out_shape=[
            jax.ShapeDtypeStruct((H, N), f32),
            jax.ShapeDtypeStruct((NW, CPT, N), f32),
            jax.ShapeDtypeStruct((1, N), f32),
            jax.ShapeDtypeStruct((1, N), f32),
        ],
    )(x, W_emb, b_emb.reshape(H, 1), W_gat,
      att_src.reshape(1, H), att_dst.reshape(1, H))

    w_e, den_part = _phase_a_call()(_phase_a_body)(
        edge_index[0], edge_index[1],
        a_s.reshape(N), a_d.reshape(N))

    (s_part,) = _phase_b_call()(_phase_b_body)(
        w_e, edge_index[0], edge_index[1], h3)

    a, v = pl.pallas_call(
        _stage3_body,
        grid=(GRID,),
        in_specs=[
            pl.BlockSpec((H, NB), lambda i: (0, i)),
            pl.BlockSpec((NW, CPT, NB), lambda i: (0, 0, i)),
            pl.BlockSpec((NW, 1, NB), lambda i: (0, 0, i)),
            full((NW, CPT, 1)), full((H, H)), full((H, H)),
            full((H, 1)), full((A, H)), full((1, A)), full((1, H)),
            full((1, 1)),
        ],
        out_specs=[
            pl.BlockSpec((NB, A), lambda i: (i, 0)),
            pl.BlockSpec((NB, 1), lambda i: (i, 0)),
        ],
        out_shape=[
            jax.ShapeDtypeStruct((N, A), f32),
            jax.ShapeDtypeStruct((N, 1), f32),
        ],
    )(localt, s_part, den_part, b_gat.reshape(NW, CPT, 1),
      W_aff[:, :H], W_aff[:, H:],
      b_aff.reshape(H, 1), W_act, b_act.reshape(1, A),
      W_val.reshape(1, H), b_val.reshape(1, 1))

    return (a, v)


# speedup vs baseline: 1.1909x; 1.1909x over previous
import functools

import jax
import jax.numpy as jnp
from jax import lax
from jax.experimental import pallas as pl
from jax.experimental.pallas import tpu as pltpu
from jax.experimental.pallas import tpu_sc as plsc

N = 10000
D = 128
H = 128
A = 32
E = 320000

NC = 2
NS = 16
L = 16
NW = NC * NS
CPT = H // NW

EPT = E // NW
CHUNK = 2000
NCH_A = EPT // CHUNK
NCH_B = E // CHUNK
GROUPS = CHUNK // L
UNROLL = 5

NB = 2048
GRID = (N + NB - 1) // NB

f32 = jnp.float32


def _stage1_body(x_ref, we_ref, be_ref, wg_ref, asr_ref, adr_ref,
                 localt_ref, h3_ref, as_ref, ad_ref):
    rowT = (((1,), (1,)), ((), ()))
    colT = (((1,), (0,)), ((), ()))
    localt = jnp.tanh(lax.dot_general(we_ref[...], x_ref[...], rowT,
                                      preferred_element_type=f32) + be_ref[...])
    ht = lax.dot_general(wg_ref[...], localt, colT, preferred_element_type=f32)
    localt_ref[...] = localt
    as_ref[...] = lax.dot_general(asr_ref[...], ht, colT,
                                  preferred_element_type=f32)
    ad_ref[...] = lax.dot_general(adr_ref[...], ht, colT,
                                  preferred_element_type=f32)
    for g in range(NW):
        h3_ref[g] = ht[CPT * g:CPT * (g + 1), :]


def _stage3_body(localt_ref, s3_ref, den_ref, bg3_ref, waffl_ref, waffr3_ref,
                 ba_ref, wact_ref, bact_ref, wval_ref, bval_ref, a_ref, v_ref):
    colT = (((1,), (0,)), ((), ()))
    den = jnp.sum(den_ref[:, 0, :], axis=0)[None, None, :]
    intra3 = jnp.tanh(s3_ref[...] / (den + 1e-16) + bg3_ref[...])
    intra = intra3.reshape(H, intra3.shape[2])
    aff_r = lax.dot_general(waffr3_ref[...], intra, colT,
                            preferred_element_type=f32)
    hidt = jnp.tanh(
        lax.dot_general(waffl_ref[...], localt_ref[...], colT,
                        preferred_element_type=f32)
        + aff_r + ba_ref[...])
    logits = lax.dot_general(hidt, wact_ref[...], (((0,), (1,)), ((), ())),
                             preferred_element_type=f32) + bact_ref[...]
    mx = jnp.max(logits, axis=1, keepdims=True)
    lse = jnp.log(jnp.sum(jnp.exp(logits - mx), axis=1, keepdims=True))
    a_ref[...] = logits - mx - lse
    v_ref[...] = lax.dot_general(hidt, wval_ref[...], (((0,), (1,)), ((), ())),
                                 preferred_element_type=f32) + bval_ref[...]


def _phase_a_body(src_hbm, dst_hbm, as_hbm, ad_hbm,
                  w_out, den_out,
                  src_v, dst_v, w_v, asv, adv, den_v):
    cid = lax.axis_index("c")
    sid = lax.axis_index("s")
    wid = sid * NC + cid
    base = wid * EPT

    pltpu.sync_copy(as_hbm, asv)
    pltpu.sync_copy(ad_hbm, adv)

    def _zden(i, _):
        den_v[pl.ds(i * L, L)] = jnp.zeros((L,), f32)
        return 0
    lax.fori_loop(0, N // L, _zden, 0)

    def _chunk(ci, _):
        pltpu.sync_copy(src_hbm.at[pl.ds(base + ci * CHUNK, CHUNK)], src_v)
        pltpu.sync_copy(dst_hbm.at[pl.ds(base + ci * CHUNK, CHUNK)], dst_v)

        def _group(g, _):
            off = g * L
            src16 = src_v[pl.ds(off, L)]
            dst16 = dst_v[pl.ds(off, L)]
            e = plsc.load_gather(asv, [src16]) + plsc.load_gather(adv, [dst16])
            e = jnp.where(e > 0, e, 0.2 * e)
            w = jnp.exp(e)
            plsc.addupdate_scatter(den_v, [dst16], w)
            w_v[pl.ds(off, L)] = w
            return 0
        lax.fori_loop(0, GROUPS, _group, 0)
        pltpu.sync_copy(w_v, w_out.at[wid, ci])
        return 0
    lax.fori_loop(0, NCH_A, _chunk, 0)

    pltpu.sync_copy(den_v, den_out.at[wid, 0])


def _phase_b_body(w_hbm, src_hbm, dst_hbm, hp_hbm,
                  s_out,
                  wa, sra, dsa, wb, srb, dsb,
                  hp0, hp1, s0, s1, s2, s3,
                  ga0, ga1, ga2, gb0, gb1, gb2):
    cid = lax.axis_index("c")
    sid = lax.axis_index("s")
    wid = sid * NC + cid
    hp = (hp0, hp1)
    sc = (s0, s1, s2, s3)
    himask = jnp.uint32(0xFFFF0000)

    for p in range(CPT // 2):
        pltpu.sync_copy(hp_hbm.at[wid, p], hp[p])

    def _zs(i, _):
        z = jnp.zeros((L,), f32)
        for c in range(CPT):
            sc[c][pl.ds(i * L, L)] = z
        return 0
    lax.fori_loop(0, N // L, _zs, 0)

    def _issue(ci, w_b, s_b, d_b, sems):
        pltpu.async_copy(w_hbm.at[ci // NCH_A, ci % NCH_A], w_b, sems[0])
        pltpu.async_copy(src_hbm.at[pl.ds(ci * CHUNK, CHUNK)], s_b, sems[1])
        pltpu.async_copy(dst_hbm.at[pl.ds(ci * CHUNK, CHUNK)], d_b, sems[2])

    def _wait(ci, w_b, s_b, d_b, sems):
        pltpu.make_async_copy(w_hbm.at[ci // NCH_A, ci % NCH_A],
                              w_b, sems[0]).wait()
        pltpu.make_async_copy(src_hbm.at[pl.ds(ci * CHUNK, CHUNK)],
                              s_b, sems[1]).wait()
        pltpu.make_async_copy(dst_hbm.at[pl.ds(ci * CHUNK, CHUNK)],
                              d_b, sems[2]).wait()

    def _process(w_b, s_b, d_b):
        def _group(g, _):
            for u in range(UNROLL):
                off = (g * UNROLL + u) * L
                w16 = w_b[pl.ds(off, L)]
                src16 = s_b[pl.ds(off, L)]
                dst16 = d_b[pl.ds(off, L)]
                for p in range(CPT // 2):
                    pv = plsc.load_gather(hp[p], [src16])
                    pu = plsc.bitcast(pv, jnp.uint32)
                    lo = plsc.bitcast(pu << 16, f32)
                    hi = plsc.bitcast(pu & himask, f32)
                    plsc.addupdate_scatter(sc[2 * p], [dst16], w16 * lo)
                    plsc.addupdate_scatter(sc[2 * p + 1], [dst16], w16 * hi)
            return 0
        lax.fori_loop(0, GROUPS // UNROLL, _group, 0)

    sems_a = (ga0, ga1, ga2)
    sems_b = (gb0, gb1, gb2)
    _issue(0, wa, sra, dsa, sems_a)

    def _pair(k, _):
        ca = 2 * k
        cb = 2 * k + 1
        _wait(ca, wa, sra, dsa, sems_a)
        _issue(cb, wb, srb, dsb, sems_b)
        _process(wa, sra, dsa)
        _wait(cb, wb, srb, dsb, sems_b)

        @pl.when(k < NCH_B // 2 - 1)
        def _():
            _issue(ca + 2, wa, sra, dsa, sems_a)
        _process(wb, srb, dsb)
        return 0
    lax.fori_loop(0, NCH_B // 2, _pair, 0)

    for c in range(CPT):
        pltpu.sync_copy(sc[c], s_out.at[wid, c])


def _phase_a_call():
    return functools.partial(
        pl.kernel,
        out_type=[jax.ShapeDtypeStruct((NW, NCH_A, CHUNK), f32),
                  jax.ShapeDtypeStruct((NW, 1, N), f32)],
        mesh=plsc.VectorSubcoreMesh(core_axis_name="c", subcore_axis_name="s",
                                    num_cores=NC, num_subcores=NS),
        scratch_types=[
            pltpu.VMEM((CHUNK,), jnp.int32),
            pltpu.VMEM((CHUNK,), jnp.int32),
            pltpu.VMEM((CHUNK,), f32),
            pltpu.VMEM((N,), f32),
            pltpu.VMEM((N,), f32),
            pltpu.VMEM((N,), f32),
        ],
        compiler_params=pltpu.CompilerParams(needs_layout_passes=False),
    )


def _phase_b_call():
    return functools.partial(
        pl.kernel,
        out_type=[jax.ShapeDtypeStruct((NW, CPT, N), f32)],
        mesh=plsc.VectorSubcoreMesh(core_axis_name="c", subcore_axis_name="s",
                                    num_cores=NC, num_subcores=NS),
        scratch_types=[
            pltpu.VMEM((CHUNK,), f32),
            pltpu.VMEM((CHUNK,), jnp.int32),
            pltpu.VMEM((CHUNK,), jnp.int32),
            pltpu.VMEM((CHUNK,), f32),
            pltpu.VMEM((CHUNK,), jnp.int32),
            pltpu.VMEM((CHUNK,), jnp.int32),
            pltpu.VMEM((N,), f32),
            pltpu.VMEM((N,), f32),
            pltpu.VMEM((N,), f32),
            pltpu.VMEM((N,), f32),
            pltpu.VMEM((N,), f32),
            pltpu.VMEM((N,), f32),
            pltpu.SemaphoreType.DMA,
            pltpu.SemaphoreType.DMA,
            pltpu.SemaphoreType.DMA,
            pltpu.SemaphoreType.DMA,
            pltpu.SemaphoreType.DMA,
            pltpu.SemaphoreType.DMA,
        ],
        compiler_params=pltpu.CompilerParams(needs_layout_passes=False),
    )


def kernel(x, edge_index, W_emb, b_emb, W_gat, att_src, att_dst, b_gat,
           W_aff, b_aff, W_act, b_act, W_val, b_val):
    full = lambda bs: pl.BlockSpec(bs, lambda i: (0,) * len(bs))

    localt, h3, a_s, a_d = pl.pallas_call(
        _stage1_body,
        grid=(GRID,),
        in_specs=[
            pl.BlockSpec((NB, D), lambda i: (i, 0)),
            full((H, D)), full((H, 1)), full((H, H)),
            full((1, H)), full((1, H)),
        ],
        out_specs=[
            pl.BlockSpec((H, NB), lambda i: (0, i)),
            pl.BlockSpec((NW, CPT, NB), lambda i: (0, 0, i)),
            pl.BlockSpec((1, NB), lambda i: (0, i)),
            pl.BlockSpec((1, NB), lambda i: (0, i)),
        ],
        out_shape=[
            jax.ShapeDtypeStruct((H, N), f32),
            jax.ShapeDtypeStruct((NW, CPT, N), f32),
            jax.ShapeDtypeStruct((1, N), f32),
            jax.ShapeDtypeStruct((1, N), f32),
        ],
    )(x, W_emb, b_emb.reshape(H, 1), W_gat,
      att_src.reshape(1, H), att_dst.reshape(1, H))

    w_e, den_part = _phase_a_call()(_phase_a_body)(
        edge_index[0], edge_index[1],
        a_s.reshape(N), a_d.reshape(N))

    hu = lax.bitcast_convert_type(
        h3.astype(jnp.bfloat16), jnp.uint16).astype(jnp.uint32)
    hp = lax.bitcast_convert_type(
        hu[:, 0::2, :] | (hu[:, 1::2, :] << 16), f32)

    (s_part,) = _phase_b_call()(_phase_b_body)(
        w_e, edge_index[0], edge_index[1], hp)

    a, v = pl.pallas_call(
        _stage3_body,
        grid=(GRID,),
        in_specs=[
            pl.BlockSpec((H, NB), lambda i: (0, i)),
            pl.BlockSpec((NW, CPT, NB), lambda i: (0, 0, i)),
            pl.BlockSpec((NW, 1, NB), lambda i: (0, 0, i)),
            full((NW, CPT, 1)), full((H, H)), full((H, H)),
            full((H, 1)), full((A, H)), full((1, A)), full((1, H)),
            full((1, 1)),
        ],
        out_specs=[
            pl.BlockSpec((NB, A), lambda i: (i, 0)),
            pl.BlockSpec((NB, 1), lambda i: (i, 0)),
        ],
        out_shape=[
            jax.ShapeDtypeStruct((N, A), f32),
            jax.ShapeDtypeStruct((N, 1), f32),
        ],
    )(localt, s_part, den_part, b_gat.reshape(NW, CPT, 1),
      W_aff[:, :H], W_aff[:, H:],
      b_aff.reshape(H, 1), W_act, b_act.reshape(1, A),
      W_val.reshape(1, H), b_val.reshape(1, 1))

    return (a, v)


# speedup vs baseline: 1.2912x; 1.0842x over previous
import functools

import jax
import jax.numpy as jnp
import numpy as np
from jax import lax
from jax.experimental import pallas as pl
from jax.experimental.pallas import tpu as pltpu
from jax.experimental.pallas import tpu_sc as plsc

N = 10000
D = 128
H = 128
A = 32
E = 320000

NC = 2
NS = 16
L = 16
NW = NC * NS
CPT = H // NW

EPT = E // NW
CHUNK = 2000
NCH_A = EPT // CHUNK
NCH_B = E // CHUNK
GROUPS = CHUNK // L
UNROLL = 5

NB = 2048
GRID = (N + NB - 1) // NB

f32 = jnp.float32

PERM = np.array([[2 * g, 64 + 2 * g, 2 * g + 1, 64 + 2 * g + 1]
                 for g in range(NW)]).reshape(-1)


def _stage1_body(x_ref, we_ref, be_ref, wg_ref, asr_ref, adr_ref,
                 localt_ref, hp_ref, as_ref, ad_ref):
    rowT = (((1,), (1,)), ((), ()))
    colT = (((1,), (0,)), ((), ()))
    localt = jnp.tanh(lax.dot_general(we_ref[...], x_ref[...], rowT,
                                      preferred_element_type=f32) + be_ref[...])
    ht = lax.dot_general(wg_ref[...], localt, colT, preferred_element_type=f32)
    localt_ref[...] = localt
    as_ref[...] = lax.dot_general(asr_ref[...], ht, colT,
                                  preferred_element_type=f32)
    ad_ref[...] = lax.dot_general(adr_ref[...], ht, colT,
                                  preferred_element_type=f32)
    u = lax.bitcast_convert_type(ht.astype(jnp.bfloat16),
                                 jnp.uint16).astype(jnp.uint32)
    pf = lax.bitcast_convert_type(u[:H // 2] | (u[H // 2:] << 16), f32)
    for g in range(NW):
        hp_ref[g] = pf[2 * g:2 * g + 2, :]


def _stage3_body(localt_ref, s3_ref, den_ref, bg3_ref, waffl_ref, waffr3_ref,
                 ba_ref, wact_ref, bact_ref, wval_ref, bval_ref, a_ref, v_ref):
    colT = (((1,), (0,)), ((), ()))
    den = jnp.sum(den_ref[:, 0, :], axis=0)[None, None, :]
    intra3 = jnp.tanh(s3_ref[...] / (den + 1e-16) + bg3_ref[...])
    intra = intra3.reshape(H, intra3.shape[2])
    aff_r = lax.dot_general(waffr3_ref[...], intra, colT,
                            preferred_element_type=f32)
    hidt = jnp.tanh(
        lax.dot_general(waffl_ref[...], localt_ref[...], colT,
                        preferred_element_type=f32)
        + aff_r + ba_ref[...])
    logits = lax.dot_general(hidt, wact_ref[...], (((0,), (1,)), ((), ())),
                             preferred_element_type=f32) + bact_ref[...]
    mx = jnp.max(logits, axis=1, keepdims=True)
    lse = jnp.log(jnp.sum(jnp.exp(logits - mx), axis=1, keepdims=True))
    a_ref[...] = logits - mx - lse
    v_ref[...] = lax.dot_general(hidt, wval_ref[...], (((0,), (1,)), ((), ())),
                                 preferred_element_type=f32) + bval_ref[...]


def _phase_a_body(src_hbm, dst_hbm, as_hbm, ad_hbm,
                  w_out, den_out,
                  src_v, dst_v, w_v, asv, adv, den_v):
    cid = lax.axis_index("c")
    sid = lax.axis_index("s")
    wid = sid * NC + cid
    base = wid * EPT

    pltpu.sync_copy(as_hbm, asv)
    pltpu.sync_copy(ad_hbm, adv)

    def _zden(i, _):
        den_v[pl.ds(i * L, L)] = jnp.zeros((L,), f32)
        return 0
    lax.fori_loop(0, N // L, _zden, 0)

    def _chunk(ci, _):
        pltpu.sync_copy(src_hbm.at[pl.ds(base + ci * CHUNK, CHUNK)], src_v)
        pltpu.sync_copy(dst_hbm.at[pl.ds(base + ci * CHUNK, CHUNK)], dst_v)

        def _group(g, _):
            off = g * L
            src16 = src_v[pl.ds(off, L)]
            dst16 = dst_v[pl.ds(off, L)]
            e = plsc.load_gather(asv, [src16]) + plsc.load_gather(adv, [dst16])
            e = jnp.where(e > 0, e, 0.2 * e)
            w = jnp.exp(e)
            plsc.addupdate_scatter(den_v, [dst16], w)
            w_v[pl.ds(off, L)] = w
            return 0
        lax.fori_loop(0, GROUPS, _group, 0)
        pltpu.sync_copy(w_v, w_out.at[wid, ci])
        return 0
    lax.fori_loop(0, NCH_A, _chunk, 0)

    pltpu.sync_copy(den_v, den_out.at[wid, 0])


def _phase_b_body(w_hbm, src_hbm, dst_hbm, hp_hbm,
                  s_out,
                  wa, sra, dsa, wb, srb, dsb,
                  hp0, hp1, s0, s1, s2, s3,
                  ga0, ga1, ga2, gb0, gb1, gb2):
    cid = lax.axis_index("c")
    sid = lax.axis_index("s")
    wid = sid * NC + cid
    hp = (hp0, hp1)
    sc = (s0, s1, s2, s3)
    himask = jnp.uint32(0xFFFF0000)

    for p in range(CPT // 2):
        pltpu.sync_copy(hp_hbm.at[wid, p], hp[p])

    def _zs(i, _):
        z = jnp.zeros((L,), f32)
        for c in range(CPT):
            sc[c][pl.ds(i * L, L)] = z
        return 0
    lax.fori_loop(0, N // L, _zs, 0)

    def _issue(ci, w_b, s_b, d_b, sems):
        pltpu.async_copy(w_hbm.at[ci // NCH_A, ci % NCH_A], w_b, sems[0])
        pltpu.async_copy(src_hbm.at[pl.ds(ci * CHUNK, CHUNK)], s_b, sems[1])
        pltpu.async_copy(dst_hbm.at[pl.ds(ci * CHUNK, CHUNK)], d_b, sems[2])

    def _wait(ci, w_b, s_b, d_b, sems):
        pltpu.make_async_copy(w_hbm.at[ci // NCH_A, ci % NCH_A],
                              w_b, sems[0]).wait()
        pltpu.make_async_copy(src_hbm.at[pl.ds(ci * CHUNK, CHUNK)],
                              s_b, sems[1]).wait()
        pltpu.make_async_copy(dst_hbm.at[pl.ds(ci * CHUNK, CHUNK)],
                              d_b, sems[2]).wait()

    def _process(w_b, s_b, d_b):
        def _group(g, _):
            for u in range(UNROLL):
                off = (g * UNROLL + u) * L
                w16 = w_b[pl.ds(off, L)]
                src16 = s_b[pl.ds(off, L)]
                dst16 = d_b[pl.ds(off, L)]
                for p in range(CPT // 2):
                    pv = plsc.load_gather(hp[p], [src16])
                    pu = plsc.bitcast(pv, jnp.uint32)
                    lo = plsc.bitcast(pu << 16, f32)
                    hi = plsc.bitcast(pu & himask, f32)
                    plsc.addupdate_scatter(sc[2 * p], [dst16], w16 * lo)
                    plsc.addupdate_scatter(sc[2 * p + 1], [dst16], w16 * hi)
            return 0
        lax.fori_loop(0, GROUPS // UNROLL, _group, 0)

    sems_a = (ga0, ga1, ga2)
    sems_b = (gb0, gb1, gb2)
    _issue(0, wa, sra, dsa, sems_a)

    def _pair(k, _):
        ca = 2 * k
        cb = 2 * k + 1
        _wait(ca, wa, sra, dsa, sems_a)
        _issue(cb, wb, srb, dsb, sems_b)
        _process(wa, sra, dsa)
        _wait(cb, wb, srb, dsb, sems_b)

        @pl.when(k < NCH_B // 2 - 1)
        def _():
            _issue(ca + 2, wa, sra, dsa, sems_a)
        _process(wb, srb, dsb)
        return 0
    lax.fori_loop(0, NCH_B // 2, _pair, 0)

    for c in range(CPT):
        pltpu.sync_copy(sc[c], s_out.at[wid, c])


def _phase_a_call():
    return functools.partial(
        pl.kernel,
        out_type=[jax.ShapeDtypeStruct((NW, NCH_A, CHUNK), f32),
                  jax.ShapeDtypeStruct((NW, 1, N), f32)],
        mesh=plsc.VectorSubcoreMesh(core_axis_name="c", subcore_axis_name="s",
                                    num_cores=NC, num_subcores=NS),
        scratch_types=[
            pltpu.VMEM((CHUNK,), jnp.int32),
            pltpu.VMEM((CHUNK,), jnp.int32),
            pltpu.VMEM((CHUNK,), f32),
            pltpu.VMEM((N,), f32),
            pltpu.VMEM((N,), f32),
            pltpu.VMEM((N,), f32),
        ],
        compiler_params=pltpu.CompilerParams(needs_layout_passes=False),
    )


def _phase_b_call():
    return functools.partial(
        pl.kernel,
        out_type=[jax.ShapeDtypeStruct((NW, CPT, N), f32)],
        mesh=plsc.VectorSubcoreMesh(core_axis_name="c", subcore_axis_name="s",
                                    num_cores=NC, num_subcores=NS),
        scratch_types=[
            pltpu.VMEM((CHUNK,), f32),
            pltpu.VMEM((CHUNK,), jnp.int32),
            pltpu.VMEM((CHUNK,), jnp.int32),
            pltpu.VMEM((CHUNK,), f32),
            pltpu.VMEM((CHUNK,), jnp.int32),
            pltpu.VMEM((CHUNK,), jnp.int32),
            pltpu.VMEM((N,), f32),
            pltpu.VMEM((N,), f32),
            pltpu.VMEM((N,), f32),
            pltpu.VMEM((N,), f32),
            pltpu.VMEM((N,), f32),
            pltpu.VMEM((N,), f32),
            pltpu.SemaphoreType.DMA,
            pltpu.SemaphoreType.DMA,
            pltpu.SemaphoreType.DMA,
            pltpu.SemaphoreType.DMA,
            pltpu.SemaphoreType.DMA,
            pltpu.SemaphoreType.DMA,
        ],
        compiler_params=pltpu.CompilerParams(needs_layout_passes=False),
    )


def kernel(x, edge_index, W_emb, b_emb, W_gat, att_src, att_dst, b_gat,
           W_aff, b_aff, W_act, b_act, W_val, b_val):
    full = lambda bs: pl.BlockSpec(bs, lambda i: (0,) * len(bs))

    localt, hp, a_s, a_d = pl.pallas_call(
        _stage1_body,
        grid=(GRID,),
        in_specs=[
            pl.BlockSpec((NB, D), lambda i: (i, 0)),
            full((H, D)), full((H, 1)), full((H, H)),
            full((1, H)), full((1, H)),
        ],
        out_specs=[
            pl.BlockSpec((H, NB), lambda i: (0, i)),
            pl.BlockSpec((NW, 2, NB), lambda i: (0, 0, i)),
            pl.BlockSpec((1, NB), lambda i: (0, i)),
            pl.BlockSpec((1, NB), lambda i: (0, i)),
        ],
        out_shape=[
            jax.ShapeDtypeStruct((H, N), f32),
            jax.ShapeDtypeStruct((NW, 2, N), f32),
            jax.ShapeDtypeStruct((1, N), f32),
            jax.ShapeDtypeStruct((1, N), f32),
        ],
    )(x, W_emb, b_emb.reshape(H, 1), W_gat,
      att_src.reshape(1, H), att_dst.reshape(1, H))

    w_e, den_part = _phase_a_call()(_phase_a_body)(
        edge_index[0], edge_index[1],
        a_s.reshape(N), a_d.reshape(N))

    (s_part,) = _phase_b_call()(_phase_b_body)(
        w_e, edge_index[0], edge_index[1], hp)

    a, v = pl.pallas_call(
        _stage3_body,
        grid=(GRID,),
        in_specs=[
            pl.BlockSpec((H, NB), lambda i: (0, i)),
            pl.BlockSpec((NW, CPT, NB), lambda i: (0, 0, i)),
            pl.BlockSpec((NW, 1, NB), lambda i: (0, 0, i)),
            full((NW, CPT, 1)), full((H, H)), full((H, H)),
            full((H, 1)), full((A, H)), full((1, A)), full((1, H)),
            full((1, 1)),
        ],
        out_specs=[
            pl.BlockSpec((NB, A), lambda i: (i, 0)),
            pl.BlockSpec((NB, 1), lambda i: (i, 0)),
        ],
        out_shape=[
            jax.ShapeDtypeStruct((N, A), f32),
            jax.ShapeDtypeStruct((N, 1), f32),
        ],
    )(localt, s_part, den_part, b_gat[PERM].reshape(NW, CPT, 1),
      W_aff[:, :H], W_aff[:, H + PERM],
      b_aff.reshape(H, 1), W_act, b_act.reshape(1, A),
      W_val.reshape(1, H), b_val.reshape(1, 1))

    return (a, v)
